# baseline (speedup 1.0000x reference)
import jax
import jax.numpy as jnp
from jax import lax
from jax.experimental import pallas as pl
from jax.experimental.pallas import tpu as pltpu
from jax.experimental.pallas import tpu_sc as plsc

BATCH = 16384
FEAT = 64
LANES = 16
BLK = 128
NUM_CORES = 2
NUM_SUBCORES = 16
NW = NUM_CORES * NUM_SUBCORES
B_PER_W = BATCH // NW
FVECS = FEAT // LANES
NBUF = 8
N_DG = B_PER_W // 16


def _center_loss_kernel(eT_hbm, lab_hbm, cT_hbm, out_hbm,
                        lab_v, eT_v, blk_v, acc_v,
                        sem_lab, sem_emb, sem_blk, sem_blk2):
    cid = lax.axis_index("c")
    sid = lax.axis_index("s")
    wid = sid * NUM_CORES + cid
    base = wid * B_PER_W

    lab_copy = pltpu.async_copy(lab_hbm.at[pl.ds(base, B_PER_W)],
                                lab_v.at[pl.ds(0, B_PER_W)], sem_lab)
    emb_copy = pltpu.async_copy(eT_hbm.at[:, pl.ds(base, B_PER_W)], eT_v,
                                sem_emb)
    lab_copy.wait()

    iota = lax.iota(jnp.int32, LANES)

    def fire(slot, lab_scalar):
        col = pl.multiple_of(jnp.bitwise_and(lab_scalar, jnp.int32(-BLK)),
                             BLK)
        pltpu.async_copy(cT_hbm.at[pl.ds(0, 32), pl.ds(col, BLK)],
                         blk_v.at[slot, pl.ds(0, 32), :], sem_blk.at[slot])
        pltpu.async_copy(cT_hbm.at[pl.ds(32, 32), pl.ds(col, BLK)],
                         blk_v.at[slot, pl.ds(32, 32), :], sem_blk2.at[slot])

    labs0 = lab_v[pl.ds(0, 16)]
    for k in range(NBUF):
        fire(k, labs0[k])
    emb_copy.wait()

    def dgroup(g, accs):
        accs = list(accs)
        labs = lab_v[pl.ds(g * 16, 16)]
        nlabs = lab_v[pl.ds((g + 1) * 16, 16)]
        for k in range(16):
            slot = k % NBUF
            item = g * 16 + k
            pltpu.make_async_copy(cT_hbm.at[pl.ds(0, 32), pl.ds(0, BLK)],
                                  blk_v.at[slot, pl.ds(0, 32), :],
                                  sem_blk.at[slot]).wait()
            pltpu.make_async_copy(cT_hbm.at[pl.ds(32, 32), pl.ds(0, BLK)],
                                  blk_v.at[slot, pl.ds(32, 32), :],
                                  sem_blk2.at[slot]).wait()
            lane_vec = jnp.broadcast_to(
                jnp.bitwise_and(labs[k], jnp.int32(BLK - 1)), (LANES,))
            item_vec = jnp.broadcast_to(jnp.int32(0) + item, (LANES,))
            for c in range(FVECS):
                rows = iota + (c * LANES)
                cv = plsc.load_gather(blk_v.at[slot], [rows, lane_vec])
                ev = plsc.load_gather(eT_v, [rows, item_vec])
                d = ev - cv
                accs[c] = accs[c] + d * d
            if k < NBUF:
                fire(slot, labs[k + NBUF])
            else:

                @pl.when(g < N_DG - 1)
                def _():
                    fire(slot, nlabs[k - NBUF])
        return tuple(accs)

    zero = jnp.zeros((LANES,), jnp.float32)
    accs = lax.fori_loop(0, N_DG, dgroup, (zero,) * FVECS)
    total = (accs[0] + accs[1]) + (accs[2] + accs[3])

    acc_v[...] = total * (1.0 / BATCH)
    pltpu.sync_copy(acc_v, out_hbm.at[cid, sid])


@jax.jit
def kernel(embeddings, labels, centers):
    labels = labels.astype(jnp.int32)
    eT = embeddings.T
    cT = centers.T
    mesh = plsc.VectorSubcoreMesh(core_axis_name="c", subcore_axis_name="s")
    out = pl.kernel(
        _center_loss_kernel,
        mesh=mesh,
        compiler_params=pltpu.CompilerParams(needs_layout_passes=False),
        out_type=jax.ShapeDtypeStruct((NUM_CORES, NUM_SUBCORES, LANES),
                                      jnp.float32),
        scratch_types=[
            pltpu.VMEM((B_PER_W + 16,), jnp.int32),
            pltpu.VMEM((FEAT, B_PER_W), jnp.float32),
            pltpu.VMEM((NBUF, FEAT, BLK), jnp.float32),
            pltpu.VMEM((LANES,), jnp.float32),
            pltpu.SemaphoreType.DMA,
            pltpu.SemaphoreType.DMA,
            pltpu.SemaphoreType.DMA((NBUF,)),
            pltpu.SemaphoreType.DMA((NBUF,)),
        ],
    )(eT, labels, cT)
    return jnp.sum(out)


# speedup vs baseline: 1.1207x; 1.1207x over previous
import jax
import jax.numpy as jnp
from jax import lax
from jax.experimental import pallas as pl
from jax.experimental.pallas import tpu as pltpu
from jax.experimental.pallas import tpu_sc as plsc

BATCH = 16384
FEAT = 64
LANES = 16
BLK = 128
NUM_CORES = 2
NUM_SUBCORES = 16
NW = NUM_CORES * NUM_SUBCORES
B_PER_W = BATCH // NW
FVECS = FEAT // LANES
NBUF = 8
N_DG = B_PER_W // 16


def _center_loss_kernel(eT_hbm, lab_hbm, cT_hbm, out_hbm,
                        lab_v, eT_v, blk_v, acc_v,
                        sem_lab, sem_emb, sem_blk):
    cid = lax.axis_index("c")
    sid = lax.axis_index("s")
    wid = sid * NUM_CORES + cid
    base = wid * B_PER_W

    lab_copy = pltpu.async_copy(lab_hbm.at[pl.ds(base, B_PER_W)],
                                lab_v.at[pl.ds(0, B_PER_W)], sem_lab)
    emb_copy = pltpu.async_copy(eT_hbm.at[:, pl.ds(base, B_PER_W)], eT_v,
                                sem_emb)
    lab_copy.wait()

    iota = lax.iota(jnp.int32, LANES)

    def fire(slot, lab_scalar):
        col = pl.multiple_of(jnp.bitwise_and(lab_scalar, jnp.int32(-BLK)),
                             BLK)
        pltpu.async_copy(cT_hbm.at[:, pl.ds(col, BLK)], blk_v.at[slot],
                         sem_blk.at[slot])

    labs0 = lab_v[pl.ds(0, 16)]
    for k in range(NBUF):
        fire(k, labs0[k])
    emb_copy.wait()

    def dgroup(g, accs):
        accs = list(accs)
        labs = lab_v[pl.ds(g * 16, 16)]
        nlabs = lab_v[pl.ds((g + 1) * 16, 16)]
        for k in range(16):
            slot = k % NBUF
            item = g * 16 + k
            pltpu.make_async_copy(cT_hbm.at[:, pl.ds(0, BLK)],
                                  blk_v.at[slot], sem_blk.at[slot]).wait()
            lane_vec = jnp.broadcast_to(
                jnp.bitwise_and(labs[k], jnp.int32(BLK - 1)), (LANES,))
            item_vec = jnp.broadcast_to(jnp.int32(0) + item, (LANES,))
            for c in range(FVECS):
                rows = iota + (c * LANES)
                cv = plsc.load_gather(blk_v.at[slot], [rows, lane_vec])
                ev = plsc.load_gather(eT_v, [rows, item_vec])
                d = ev - cv
                accs[c] = accs[c] + d * d
            if k < NBUF:
                fire(slot, labs[k + NBUF])
            else:

                @pl.when(g < N_DG - 1)
                def _():
                    fire(slot, nlabs[k - NBUF])
        return tuple(accs)

    zero = jnp.zeros((LANES,), jnp.float32)
    accs = lax.fori_loop(0, N_DG, dgroup, (zero,) * FVECS)
    total = (accs[0] + accs[1]) + (accs[2] + accs[3])

    acc_v[...] = total * (1.0 / BATCH)
    pltpu.sync_copy(acc_v, out_hbm.at[cid, sid])


@jax.jit
def kernel(embeddings, labels, centers):
    labels = labels.astype(jnp.int32)
    eT = embeddings.T
    cT = centers.T
    mesh = plsc.VectorSubcoreMesh(core_axis_name="c", subcore_axis_name="s")
    out = pl.kernel(
        _center_loss_kernel,
        mesh=mesh,
        compiler_params=pltpu.CompilerParams(needs_layout_passes=False),
        out_type=jax.ShapeDtypeStruct((NUM_CORES, NUM_SUBCORES, LANES),
                                      jnp.float32),
        scratch_types=[
            pltpu.VMEM((B_PER_W + 16,), jnp.int32),
            pltpu.VMEM((FEAT, B_PER_W), jnp.float32),
            pltpu.VMEM((NBUF, FEAT, BLK), jnp.float32),
            pltpu.VMEM((LANES,), jnp.float32),
            pltpu.SemaphoreType.DMA,
            pltpu.SemaphoreType.DMA,
            pltpu.SemaphoreType.DMA((NBUF,)),
        ],
    )(eT, labels, cT)
    return jnp.sum(out)
